# baseline (speedup 1.0000x reference)
import jax
import jax.numpy as jnp
from jax import lax
from jax.experimental import pallas as pl
from jax.experimental.pallas import tpu as pltpu
from jax.experimental.pallas import tpu_sc as plsc

NUM_EMBEDDINGS = 1000
EMBED_DIM = 7
CONT_DIM = 6
OUT_DIM = EMBED_DIM + CONT_DIM

NC = 2
NS = 16
NW = NC * NS
LANES = 16

B = 16384
L = 200
TAB_STRIDE = 1024

BW = B // NW
NLB = L // 8
XW = BW // LANES


def _sc_body(idx_hbm, cont_hbm, tab_hbm, out_hbm, tab_v, idx_v, emb_v,
             sem_cont, sem_idx, sem_out):
  wid = lax.axis_index("s") * NC + lax.axis_index("c")
  b0 = wid * BW

  cont_cp = pltpu.make_async_copy(
      cont_hbm.at[:, :, pl.ds(b0, BW)],
      out_hbm.at[pl.ds(EMBED_DIM, CONT_DIM), :, pl.ds(b0, BW)],
      sem_cont)
  cont_cp.start()

  pltpu.sync_copy(tab_hbm, tab_v)

  def idx_copy(li, s):
    return pltpu.make_async_copy(
        idx_hbm.at[pl.ds(li * 8, 8), pl.ds(b0, BW)], idx_v.at[s],
        sem_idx.at[s])

  def out_copy(li, s):
    return pltpu.make_async_copy(
        emb_v.at[s],
        out_hbm.at[pl.ds(0, EMBED_DIM), pl.ds(li * 8, 8), pl.ds(b0, BW)],
        sem_out.at[s])

  def gather_block(s):
    def gcol(x, _):
      xoff = x * LANES
      for r in range(8):
        iv = idx_v[s, r, pl.ds(xoff, LANES)]
        for c in range(EMBED_DIM):
          vals = plsc.load_gather(tab_v, [iv + (c * TAB_STRIDE)])
          emb_v[s, c, r, pl.ds(xoff, LANES)] = vals
      return 0

    lax.fori_loop(0, XW, gcol, 0)

  idx_copy(0, 0).start()
  for li in (0, 1):
    s = li % 2
    idx_copy(li + 1, 1 - s).start()
    idx_copy(li, s).wait()
    gather_block(s)
    out_copy(li, s).start()

  def pair_body(p, _):
    for s in (0, 1):
      li = p * 2 + s
      idx_copy(li + 1, 1 - s).start()
      idx_copy(li, s).wait()
      out_copy(li, s).wait()
      gather_block(s)
      out_copy(li, s).start()
    return 0

  lax.fori_loop(1, (NLB - 1) // 2, pair_body, 0)

  li = NLB - 1
  idx_copy(li, 0).wait()
  out_copy(li, 0).wait()
  gather_block(0)
  out_copy(li, 0).start()

  out_copy(NLB - 2, 1).wait()
  out_copy(NLB - 1, 0).wait()
  cont_cp.wait()


@jax.jit
def _run(idx_t, cont_t, tab_cols):
  mesh = plsc.VectorSubcoreMesh(
      core_axis_name="c", subcore_axis_name="s", num_cores=NC,
      num_subcores=NS)
  f = pl.kernel(
      _sc_body,
      out_type=jax.ShapeDtypeStruct((OUT_DIM, L, B), jnp.float32),
      mesh=mesh,
      compiler_params=pltpu.CompilerParams(
          needs_layout_passes=False, use_tc_tiling_on_sc=True),
      scratch_types=[
          pltpu.VMEM((EMBED_DIM * TAB_STRIDE,), jnp.float32),
          pltpu.VMEM((2, 8, BW), jnp.int32),
          pltpu.VMEM((2, EMBED_DIM, 8, BW), jnp.float32),
          pltpu.SemaphoreType.DMA,
          pltpu.SemaphoreType.DMA((2,)),
          pltpu.SemaphoreType.DMA((2,)),
      ],
  )
  return f(idx_t, cont_t, tab_cols)


def kernel(SidebarAssetName, SidebarContinuous, buildable_embedding_weight):
  idx_t = jnp.transpose(SidebarAssetName.astype(jnp.int32), (1, 0))
  cont_t = jnp.transpose(SidebarContinuous, (2, 1, 0))
  tab_cols = jnp.zeros((EMBED_DIM, TAB_STRIDE), jnp.float32)
  tab_cols = tab_cols.at[:, :NUM_EMBEDDINGS].set(
      buildable_embedding_weight.T).reshape(EMBED_DIM * TAB_STRIDE)
  out = _run(idx_t, cont_t, tab_cols)
  return jnp.transpose(out, (2, 1, 0))


# speedup vs baseline: 10.7022x; 10.7022x over previous
import jax
import jax.numpy as jnp
from jax import lax
from jax.experimental import pallas as pl
from jax.experimental.pallas import tpu as pltpu
from jax.experimental.pallas import tpu_sc as plsc

NUM_EMBEDDINGS = 1000
EMBED_DIM = 7
CONT_DIM = 6
OUT_DIM = EMBED_DIM + CONT_DIM

NC = 2
NS = 16
NW = NC * NS
LANES = 16

B = 16384
L = 200
TAB_STRIDE = 1024

BW = B // NW
NLB = L // 8
XW = BW // LANES


def _sc_body(idx_hbm, cont_hbm, tab_hbm, out_hbm, tab_v, idx_v, slab_v,
             sem_cont, sem_idx, sem_out):
  wid = lax.axis_index("s") * NC + lax.axis_index("c")
  b0 = wid * BW

  pltpu.sync_copy(tab_hbm, tab_v)

  def idx_copy(li, s):
    return pltpu.make_async_copy(
        idx_hbm.at[pl.ds(li * 8, 8), pl.ds(b0, BW)], idx_v.at[s],
        sem_idx.at[s])

  def cont_copy(li, s):
    return pltpu.make_async_copy(
        cont_hbm.at[:, pl.ds(li * 8, 8), pl.ds(b0, BW)],
        slab_v.at[s, pl.ds(EMBED_DIM, CONT_DIM)], sem_cont.at[s])

  def out_copy(li, s):
    return pltpu.make_async_copy(
        slab_v.at[s],
        out_hbm.at[:, pl.ds(li * 8, 8), pl.ds(b0, BW)], sem_out.at[s])

  def gather_block(s):
    def gcol(x, _):
      xoff = x * LANES
      for r in range(8):
        iv = idx_v[s, r, pl.ds(xoff, LANES)]
        for c in range(EMBED_DIM):
          vals = plsc.load_gather(tab_v, [iv + (c * TAB_STRIDE)])
          slab_v[s, c, r, pl.ds(xoff, LANES)] = vals
      return 0

    lax.fori_loop(0, XW, gcol, 0)

  idx_copy(0, 0).start()
  for li in (0, 1):
    s = li % 2
    cont_copy(li, s).start()
    idx_copy(li + 1, 1 - s).start()
    idx_copy(li, s).wait()
    gather_block(s)
    cont_copy(li, s).wait()
    out_copy(li, s).start()

  def pair_body(p, _):
    for s in (0, 1):
      li = p * 2 + s
      out_copy(li, s).wait()
      cont_copy(li, s).start()
      idx_copy(li + 1, 1 - s).start()
      idx_copy(li, s).wait()
      gather_block(s)
      cont_copy(li, s).wait()
      out_copy(li, s).start()
    return 0

  lax.fori_loop(1, (NLB - 1) // 2, pair_body, 0)

  li = NLB - 1
  out_copy(li, 0).wait()
  cont_copy(li, 0).start()
  idx_copy(li, 0).wait()
  gather_block(0)
  cont_copy(li, 0).wait()
  out_copy(li, 0).start()

  out_copy(NLB - 2, 1).wait()
  out_copy(NLB - 1, 0).wait()


@jax.jit
def _run(idx_t, cont_t, tab_cols):
  mesh = plsc.VectorSubcoreMesh(
      core_axis_name="c", subcore_axis_name="s", num_cores=NC,
      num_subcores=NS)
  f = pl.kernel(
      _sc_body,
      out_type=jax.ShapeDtypeStruct((OUT_DIM, L, B), jnp.float32),
      mesh=mesh,
      compiler_params=pltpu.CompilerParams(
          needs_layout_passes=False, use_tc_tiling_on_sc=True),
      scratch_types=[
          pltpu.VMEM((EMBED_DIM * TAB_STRIDE,), jnp.float32),
          pltpu.VMEM((2, 8, BW), jnp.int32),
          pltpu.VMEM((2, OUT_DIM, 8, BW), jnp.float32),
          pltpu.SemaphoreType.DMA((2,)),
          pltpu.SemaphoreType.DMA((2,)),
          pltpu.SemaphoreType.DMA((2,)),
      ],
  )
  return f(idx_t, cont_t, tab_cols)


def kernel(SidebarAssetName, SidebarContinuous, buildable_embedding_weight):
  idx_t = jnp.transpose(SidebarAssetName.astype(jnp.int32), (1, 0))
  cont_t = jnp.transpose(SidebarContinuous, (2, 1, 0))
  tab_cols = jnp.zeros((EMBED_DIM, TAB_STRIDE), jnp.float32)
  tab_cols = tab_cols.at[:, :NUM_EMBEDDINGS].set(
      buildable_embedding_weight.T).reshape(EMBED_DIM * TAB_STRIDE)
  out = _run(idx_t, cont_t, tab_cols)
  return jnp.transpose(out, (2, 1, 0))
